# baseline (speedup 1.0000x reference)
import functools

import jax
import jax.numpy as jnp
from jax import lax
from jax.experimental import pallas as pl
from jax.experimental.pallas import tpu as pltpu
from jax.experimental.pallas import tpu_sc as plsc

D_MODEL = 768
NUM_EXPERTS = 8
D_FF = 2048
TOP_K = 2
T_TOKENS = 2048

TILE = 512
N_PAIRS = T_TOKENS * TOP_K
G_MAX = N_PAIRS // TILE + NUM_EXPERTS
NP = G_MAX * TILE
NC, NS = 2, 16
NW = NC * NS
TOK_W = T_TOKENS // NW


def _dot_t(a, b):
    return jax.lax.dot_general(
        a, b, dimension_numbers=(((1,), (1,)), ((), ())),
        preferred_element_type=jnp.float32)


def _top2(xt, rw):
    logits = _dot_t(xt, rw)
    ii = jax.lax.broadcasted_iota(jnp.int32, logits.shape, 1)
    m1 = jnp.max(logits, axis=1, keepdims=True)
    i1 = jnp.min(jnp.where(logits == m1, ii, NUM_EXPERTS), axis=1, keepdims=True)
    l2 = jnp.where(ii == i1, -jnp.inf, logits)
    m2 = jnp.max(l2, axis=1, keepdims=True)
    i2 = jnp.min(jnp.where(l2 == m2, ii, NUM_EXPERTS), axis=1, keepdims=True)
    r = jnp.exp(m2 - m1)
    p1 = 1.0 / (1.0 + r)
    return i1, i2, p1, r * p1


def _router_body(x_ref, rw_ref, i1_ref, i2_ref):
    i1, i2, _, _ = _top2(x_ref[...], rw_ref[...])
    i1_ref[...] = i1
    i2_ref[...] = i2


def _run_router(xt, router_w):
    T = xt.shape[0]
    shp = [jax.ShapeDtypeStruct((T, 1), jnp.int32),
           jax.ShapeDtypeStruct((T, 1), jnp.int32)]
    return pl.pallas_call(_router_body, out_shape=shp)(xt, router_w)


@functools.cache
def _sc_mesh():
    return plsc.VectorSubcoreMesh(
        core_axis_name="c", subcore_axis_name="s",
        num_cores=NC, num_subcores=NS)


@functools.cache
def _make_sc_dispatch():
    @functools.partial(
        pl.kernel, mesh=_sc_mesh(),
        out_type=jax.ShapeDtypeStruct((NP, D_MODEL), jnp.float32),
        scratch_types=[
            pltpu.VMEM((TOK_W,), jnp.int32),
            pltpu.VMEM((TOK_W,), jnp.int32),
            pltpu.VMEM((TOK_W, D_MODEL), jnp.float32),
            pltpu.SemaphoreType.DMA,
            pltpu.SemaphoreType.DMA,
        ],
    )
    def _sc_dispatch(x_hbm, pos0_hbm, pos1_hbm, xs_hbm, i0, i1, xbuf, s0, s1):
        wid = lax.axis_index("s") * NC + lax.axis_index("c")
        t0 = wid * TOK_W
        pltpu.sync_copy(pos0_hbm.at[pl.ds(t0, TOK_W)], i0)
        pltpu.sync_copy(pos1_hbm.at[pl.ds(t0, TOK_W)], i1)
        pltpu.sync_copy(x_hbm.at[pl.ds(t0, TOK_W)], xbuf)
        c0 = pltpu.async_copy(xbuf, xs_hbm.at[i0], s0)
        c1 = pltpu.async_copy(xbuf, xs_hbm.at[i1], s1)
        c0.wait()
        c1.wait()

    return _sc_dispatch


def _dispatch_rows(xt, pos0, pos1):
    return _make_sc_dispatch()(xt, pos0, pos1)


def _ffn_body(m_ref, xs_ref, rw_ref, w1_ref, w2_ref, w3_ref, ys_ref):
    pid = pl.program_id(0)

    @pl.when(pid < m_ref[0])
    def _():
        e = m_ref[1 + pid]
        xt = xs_ref[...]
        i1, i2, p1, p2 = _top2(xt, rw_ref[...])
        w = p1 * (i1 == e) + p2 * (i2 == e)
        a = _dot_t(xt, w1_ref[0])
        b = _dot_t(xt, w3_ref[0])
        h = (a * jax.nn.sigmoid(a)) * b
        ys_ref[...] = _dot_t(h, w2_ref[0]) * w


def _run_ffn(meta, xs, router_w, W1, W2, W3):
    grid_spec = pltpu.PrefetchScalarGridSpec(
        num_scalar_prefetch=1,
        grid=(G_MAX,),
        in_specs=[
            pl.BlockSpec((TILE, D_MODEL), lambda g, m: (g, 0)),
            pl.BlockSpec((NUM_EXPERTS, D_MODEL), lambda g, m: (0, 0)),
            pl.BlockSpec((1, D_FF, D_MODEL), lambda g, m: (m[1 + g], 0, 0)),
            pl.BlockSpec((1, D_MODEL, D_FF), lambda g, m: (m[1 + g], 0, 0)),
            pl.BlockSpec((1, D_FF, D_MODEL), lambda g, m: (m[1 + g], 0, 0)),
        ],
        out_specs=pl.BlockSpec((TILE, D_MODEL), lambda g, m: (g, 0)),
    )
    return pl.pallas_call(
        _ffn_body,
        grid_spec=grid_spec,
        out_shape=jax.ShapeDtypeStruct((NP, D_MODEL), jnp.float32),
        compiler_params=pltpu.CompilerParams(vmem_limit_bytes=100 * 1024 * 1024),
    )(meta, xs, router_w, W1, W2, W3)


@functools.cache
def _make_sc_combine():
    @functools.partial(
        pl.kernel, mesh=_sc_mesh(),
        out_type=jax.ShapeDtypeStruct((T_TOKENS, D_MODEL), jnp.float32),
        scratch_types=[
            pltpu.VMEM((TOK_W,), jnp.int32),
            pltpu.VMEM((TOK_W,), jnp.int32),
            pltpu.VMEM((TOK_W, D_MODEL), jnp.float32),
            pltpu.VMEM((TOK_W, D_MODEL), jnp.float32),
            pltpu.SemaphoreType.DMA,
            pltpu.SemaphoreType.DMA,
            pltpu.SemaphoreType.DMA,
            pltpu.SemaphoreType.DMA,
        ],
    )
    def _sc_combine(ys_hbm, pos0_hbm, pos1_hbm, out_hbm, i0, i1, b0, b1,
                    s00, s01, s10, s11):
        wid = lax.axis_index("s") * NC + lax.axis_index("c")
        t0 = wid * TOK_W
        H = TOK_W // 2
        pltpu.sync_copy(pos0_hbm.at[pl.ds(t0, TOK_W)], i0)
        pltpu.sync_copy(pos1_hbm.at[pl.ds(t0, TOK_W)], i1)
        c00 = pltpu.async_copy(ys_hbm.at[i0.at[pl.ds(0, H)]], b0.at[pl.ds(0, H)], s00)
        c01 = pltpu.async_copy(ys_hbm.at[i1.at[pl.ds(0, H)]], b1.at[pl.ds(0, H)], s01)
        c10 = pltpu.async_copy(ys_hbm.at[i0.at[pl.ds(H, H)]], b0.at[pl.ds(H, H)], s10)
        c11 = pltpu.async_copy(ys_hbm.at[i1.at[pl.ds(H, H)]], b1.at[pl.ds(H, H)], s11)

        def _row(r, carry):
            for c in range(D_MODEL // 16):
                sl = pl.ds(c * 16, 16)
                b0[r, sl] = b0[r, sl] + b1[r, sl]
            return carry

        c00.wait()
        c01.wait()
        lax.fori_loop(0, H, _row, 0)
        pltpu.sync_copy(b0.at[pl.ds(0, H)], out_hbm.at[pl.ds(t0, H)])
        c10.wait()
        c11.wait()
        lax.fori_loop(H, TOK_W, _row, 0)
        pltpu.sync_copy(b0.at[pl.ds(H, H)], out_hbm.at[pl.ds(t0 + H, H)])

    return _sc_combine


def _combine_rows(ys, pos0, pos1):
    return _make_sc_combine()(ys, pos0, pos1)


def kernel(x, router_w, W1, W2, W3):
    B, S, D = x.shape
    T = B * S
    xt = x.reshape(T, D)

    i1, i2 = _run_router(xt, router_w)

    er = jnp.arange(NUM_EXPERTS, dtype=jnp.int32)[None, :]
    oh1 = (i1 == er).astype(jnp.int32)
    oh2 = (i2 == er).astype(jnp.int32)
    both = oh1 + oh2
    csum = jnp.cumsum(both, axis=0)
    excl = csum - both
    counts = csum[-1]
    padded_tiles = (counts + TILE - 1) // TILE
    bounds = jnp.cumsum(padded_tiles)
    pad_off = (jnp.concatenate([jnp.zeros((1,), jnp.int32), bounds[:-1]])
               * TILE)[None, :]
    pos0 = jnp.sum((excl + pad_off) * oh1, axis=1)
    pos1 = jnp.sum((excl + pad_off) * oh2, axis=1)
    tile_eid = jnp.minimum(
        jnp.sum(jnp.arange(G_MAX, dtype=jnp.int32)[:, None] >= bounds[None, :],
                axis=1).astype(jnp.int32),
        NUM_EXPERTS - 1)
    meta = jnp.concatenate([bounds[-1:], tile_eid])

    xs = _dispatch_rows(xt, pos0, pos1)
    ys = _run_ffn(meta, xs, router_w, W1, W2, W3)
    out = _combine_rows(ys, pos0, pos1)
    return out.reshape(B, S, D)


# speedup vs baseline: 1.0118x; 1.0118x over previous
import functools

import jax
import jax.numpy as jnp
from jax import lax
from jax.experimental import pallas as pl
from jax.experimental.pallas import tpu as pltpu
from jax.experimental.pallas import tpu_sc as plsc

D_MODEL = 768
NUM_EXPERTS = 8
D_FF = 2048
TOP_K = 2
T_TOKENS = 2048

TILE = 512
N_PAIRS = T_TOKENS * TOP_K
G_MAX = N_PAIRS // TILE + NUM_EXPERTS
NP = G_MAX * TILE
NC, NS = 2, 16
NW = NC * NS
TOK_W = T_TOKENS // NW


def _dot_t(a, b):
    return jax.lax.dot_general(
        a, b, dimension_numbers=(((1,), (1,)), ((), ())),
        preferred_element_type=jnp.float32)


def _top2(xt, rw):
    logits = _dot_t(xt, rw)
    ii = jax.lax.broadcasted_iota(jnp.int32, logits.shape, 1)
    m1 = jnp.max(logits, axis=1, keepdims=True)
    i1 = jnp.min(jnp.where(logits == m1, ii, NUM_EXPERTS), axis=1, keepdims=True)
    l2 = jnp.where(ii == i1, -jnp.inf, logits)
    m2 = jnp.max(l2, axis=1, keepdims=True)
    i2 = jnp.min(jnp.where(l2 == m2, ii, NUM_EXPERTS), axis=1, keepdims=True)
    r = jnp.exp(m2 - m1)
    p1 = 1.0 / (1.0 + r)
    return i1, i2, p1, r * p1


_CB = 128
_NB = T_TOKENS // _CB


def _router_body(x_ref, rw_ref, pos0_ref, pos1_ref, meta_ref):
    i1, i2, _, _ = _top2(x_ref[...], rw_ref[...])
    er = jax.lax.broadcasted_iota(jnp.int32, (T_TOKENS, NUM_EXPERTS), 1)
    oh1 = (i1 == er).astype(jnp.float32)
    oh2 = (i2 == er).astype(jnp.float32)
    both = oh1 + oh2

    row = jax.lax.broadcasted_iota(jnp.int32, (_CB, _CB), 0)
    col = jax.lax.broadcasted_iota(jnp.int32, (_CB, _CB), 1)
    stril = (row > col).astype(jnp.float32)
    carry = jnp.zeros((1, NUM_EXPERTS), jnp.float32)
    parts = []
    for b in range(_NB):
        blk = both[b * _CB:(b + 1) * _CB, :]
        parts.append(jax.lax.dot(stril, blk,
                                 precision=jax.lax.Precision.HIGHEST) + carry)
        carry = carry + jnp.sum(blk, axis=0, keepdims=True)
    excl = jnp.concatenate(parts, axis=0)

    counts = carry
    padded_tiles = jnp.floor((counts + (TILE - 1)) * (1.0 / TILE))
    r8 = jax.lax.broadcasted_iota(jnp.int32, (NUM_EXPERTS, NUM_EXPERTS), 0)
    c8 = jax.lax.broadcasted_iota(jnp.int32, (NUM_EXPERTS, NUM_EXPERTS), 1)
    triu = (r8 <= c8).astype(jnp.float32)
    bounds = jax.lax.dot(padded_tiles, triu,
                         precision=jax.lax.Precision.HIGHEST)
    pad_off = (bounds - padded_tiles) * TILE
    slot = excl + pad_off
    pos0_ref[...] = jnp.sum(slot * oh1, axis=1, keepdims=True).astype(jnp.int32)
    pos1_ref[...] = jnp.sum(slot * oh2, axis=1, keepdims=True).astype(jnp.int32)

    n_real = jnp.sum(padded_tiles)
    gi = jax.lax.broadcasted_iota(jnp.int32, (_CB, NUM_EXPERTS), 0)
    eid = jnp.minimum(
        jnp.sum(((gi - 1).astype(jnp.float32) >= bounds).astype(jnp.float32),
                axis=1, keepdims=True),
        NUM_EXPERTS - 1.0)
    l0 = jax.lax.broadcasted_iota(jnp.int32, (_CB, 1), 0)
    meta_ref[...] = jnp.where(l0 == 0, n_real, eid).astype(jnp.int32)


def _run_router(xt, router_w):
    T = xt.shape[0]
    shp = [jax.ShapeDtypeStruct((T, 1), jnp.int32),
           jax.ShapeDtypeStruct((T, 1), jnp.int32),
           jax.ShapeDtypeStruct((_CB, 1), jnp.int32)]
    return pl.pallas_call(_router_body, out_shape=shp)(xt, router_w)


@functools.cache
def _sc_mesh():
    return plsc.VectorSubcoreMesh(
        core_axis_name="c", subcore_axis_name="s",
        num_cores=NC, num_subcores=NS)


@functools.cache
def _make_sc_dispatch():
    @functools.partial(
        pl.kernel, mesh=_sc_mesh(),
        out_type=jax.ShapeDtypeStruct((NP, D_MODEL), jnp.float32),
        scratch_types=[
            pltpu.VMEM((TOK_W,), jnp.int32),
            pltpu.VMEM((TOK_W,), jnp.int32),
            pltpu.VMEM((TOK_W, D_MODEL), jnp.float32),
            pltpu.SemaphoreType.DMA,
            pltpu.SemaphoreType.DMA,
        ],
    )
    def _sc_dispatch(x_hbm, pos0_hbm, pos1_hbm, xs_hbm, i0, i1, xbuf, s0, s1):
        wid = lax.axis_index("s") * NC + lax.axis_index("c")
        t0 = wid * TOK_W
        pltpu.sync_copy(pos0_hbm.at[pl.ds(t0, TOK_W)], i0)
        pltpu.sync_copy(pos1_hbm.at[pl.ds(t0, TOK_W)], i1)
        pltpu.sync_copy(x_hbm.at[pl.ds(t0, TOK_W)], xbuf)
        c0 = pltpu.async_copy(xbuf, xs_hbm.at[i0], s0)
        c1 = pltpu.async_copy(xbuf, xs_hbm.at[i1], s1)
        c0.wait()
        c1.wait()

    return _sc_dispatch


def _dispatch_rows(xt, pos0, pos1):
    return _make_sc_dispatch()(xt, pos0, pos1)


def _ffn_body(m_ref, xs_ref, rw_ref, w1_ref, w2_ref, w3_ref, ys_ref):
    pid = pl.program_id(0)

    @pl.when(pid < m_ref[0])
    def _():
        e = m_ref[1 + pid]
        xt = xs_ref[...]
        i1, i2, p1, p2 = _top2(xt, rw_ref[...])
        w = p1 * (i1 == e) + p2 * (i2 == e)
        a = _dot_t(xt, w1_ref[0])
        b = _dot_t(xt, w3_ref[0])
        h = (a * jax.nn.sigmoid(a)) * b
        ys_ref[...] = _dot_t(h, w2_ref[0]) * w


def _run_ffn(meta, xs, router_w, W1, W2, W3):
    grid_spec = pltpu.PrefetchScalarGridSpec(
        num_scalar_prefetch=1,
        grid=(G_MAX,),
        in_specs=[
            pl.BlockSpec((TILE, D_MODEL), lambda g, m: (g, 0)),
            pl.BlockSpec((NUM_EXPERTS, D_MODEL), lambda g, m: (0, 0)),
            pl.BlockSpec((1, D_FF, D_MODEL), lambda g, m: (m[1 + g], 0, 0)),
            pl.BlockSpec((1, D_MODEL, D_FF), lambda g, m: (m[1 + g], 0, 0)),
            pl.BlockSpec((1, D_FF, D_MODEL), lambda g, m: (m[1 + g], 0, 0)),
        ],
        out_specs=pl.BlockSpec((TILE, D_MODEL), lambda g, m: (g, 0)),
    )
    return pl.pallas_call(
        _ffn_body,
        grid_spec=grid_spec,
        out_shape=jax.ShapeDtypeStruct((NP, D_MODEL), jnp.float32),
        compiler_params=pltpu.CompilerParams(vmem_limit_bytes=100 * 1024 * 1024),
    )(meta, xs, router_w, W1, W2, W3)


@functools.cache
def _make_sc_combine():
    @functools.partial(
        pl.kernel, mesh=_sc_mesh(),
        out_type=jax.ShapeDtypeStruct((T_TOKENS, D_MODEL), jnp.float32),
        scratch_types=[
            pltpu.VMEM((TOK_W,), jnp.int32),
            pltpu.VMEM((TOK_W,), jnp.int32),
            pltpu.VMEM((TOK_W, D_MODEL), jnp.float32),
            pltpu.VMEM((TOK_W, D_MODEL), jnp.float32),
            pltpu.SemaphoreType.DMA,
            pltpu.SemaphoreType.DMA,
            pltpu.SemaphoreType.DMA,
            pltpu.SemaphoreType.DMA,
        ],
    )
    def _sc_combine(ys_hbm, pos0_hbm, pos1_hbm, out_hbm, i0, i1, b0, b1,
                    s00, s01, s10, s11):
        wid = lax.axis_index("s") * NC + lax.axis_index("c")
        t0 = wid * TOK_W
        H = TOK_W // 2
        pltpu.sync_copy(pos0_hbm.at[pl.ds(t0, TOK_W)], i0)
        pltpu.sync_copy(pos1_hbm.at[pl.ds(t0, TOK_W)], i1)
        c00 = pltpu.async_copy(ys_hbm.at[i0.at[pl.ds(0, H)]], b0.at[pl.ds(0, H)], s00)
        c01 = pltpu.async_copy(ys_hbm.at[i1.at[pl.ds(0, H)]], b1.at[pl.ds(0, H)], s01)
        c10 = pltpu.async_copy(ys_hbm.at[i0.at[pl.ds(H, H)]], b0.at[pl.ds(H, H)], s10)
        c11 = pltpu.async_copy(ys_hbm.at[i1.at[pl.ds(H, H)]], b1.at[pl.ds(H, H)], s11)

        def _row(r, carry):
            for c in range(D_MODEL // 16):
                sl = pl.ds(c * 16, 16)
                b0[r, sl] = b0[r, sl] + b1[r, sl]
            return carry

        c00.wait()
        c01.wait()
        lax.fori_loop(0, H, _row, 0)
        pltpu.sync_copy(b0.at[pl.ds(0, H)], out_hbm.at[pl.ds(t0, H)])
        c10.wait()
        c11.wait()
        lax.fori_loop(H, TOK_W, _row, 0)
        pltpu.sync_copy(b0.at[pl.ds(H, H)], out_hbm.at[pl.ds(t0 + H, H)])

    return _sc_combine


def _combine_rows(ys, pos0, pos1):
    return _make_sc_combine()(ys, pos0, pos1)


def kernel(x, router_w, W1, W2, W3):
    B, S, D = x.shape
    T = B * S
    xt = x.reshape(T, D)

    pos0, pos1, meta_col = _run_router(xt, router_w)
    pos0 = pos0.reshape(T)
    pos1 = pos1.reshape(T)
    meta = meta_col[:G_MAX + 1, 0]

    xs = _dispatch_rows(xt, pos0, pos1)
    ys = _run_ffn(meta, xs, router_w, W1, W2, W3)
    out = _combine_rows(ys, pos0, pos1)
    return out.reshape(B, S, D)


# speedup vs baseline: 1.0230x; 1.0111x over previous
import functools

import jax
import jax.numpy as jnp
from jax import lax
from jax.experimental import pallas as pl
from jax.experimental.pallas import tpu as pltpu
from jax.experimental.pallas import tpu_sc as plsc

D_MODEL = 768
NUM_EXPERTS = 8
D_FF = 2048
TOP_K = 2
T_TOKENS = 2048

TILE = 512
N_PAIRS = T_TOKENS * TOP_K
G_MAX = N_PAIRS // TILE + NUM_EXPERTS
NP = G_MAX * TILE
NC, NS = 2, 16
NW = NC * NS
TOK_W = T_TOKENS // NW


def _dot_t(a, b):
    return jax.lax.dot_general(
        a, b, dimension_numbers=(((1,), (1,)), ((), ())),
        preferred_element_type=jnp.float32)


def _top2(xt, rw):
    logits = _dot_t(xt, rw)
    ii = jax.lax.broadcasted_iota(jnp.int32, logits.shape, 1)
    m1 = jnp.max(logits, axis=1, keepdims=True)
    i1 = jnp.min(jnp.where(logits == m1, ii, NUM_EXPERTS), axis=1, keepdims=True)
    l2 = jnp.where(ii == i1, -jnp.inf, logits)
    m2 = jnp.max(l2, axis=1, keepdims=True)
    i2 = jnp.min(jnp.where(l2 == m2, ii, NUM_EXPERTS), axis=1, keepdims=True)
    r = jnp.exp(m2 - m1)
    p1 = 1.0 / (1.0 + r)
    return i1, i2, p1, r * p1


_CB = 128
_NB = T_TOKENS // _CB


def _router_body(x_ref, rw_ref, pos0_ref, pos1_ref, meta_ref):
    i1, i2, _, _ = _top2(x_ref[...], rw_ref[...])
    er = jax.lax.broadcasted_iota(jnp.int32, (T_TOKENS, NUM_EXPERTS), 1)
    oh1 = (i1 == er).astype(jnp.float32)
    oh2 = (i2 == er).astype(jnp.float32)
    both = oh1 + oh2

    row = jax.lax.broadcasted_iota(jnp.int32, (_CB, _CB), 0)
    col = jax.lax.broadcasted_iota(jnp.int32, (_CB, _CB), 1)
    stril = (row > col).astype(jnp.float32)
    carry = jnp.zeros((1, NUM_EXPERTS), jnp.float32)
    parts = []
    for b in range(_NB):
        blk = both[b * _CB:(b + 1) * _CB, :]
        parts.append(jax.lax.dot(stril, blk,
                                 precision=jax.lax.Precision.DEFAULT) + carry)
        carry = carry + jnp.sum(blk, axis=0, keepdims=True)
    excl = jnp.concatenate(parts, axis=0)

    counts = carry
    padded_tiles = jnp.floor((counts + (TILE - 1)) * (1.0 / TILE))
    r8 = jax.lax.broadcasted_iota(jnp.int32, (NUM_EXPERTS, NUM_EXPERTS), 0)
    c8 = jax.lax.broadcasted_iota(jnp.int32, (NUM_EXPERTS, NUM_EXPERTS), 1)
    triu = (r8 <= c8).astype(jnp.float32)
    bounds = jax.lax.dot(padded_tiles, triu,
                         precision=jax.lax.Precision.DEFAULT)
    pad_off = (bounds - padded_tiles) * TILE
    slot = excl + pad_off
    pos0_ref[...] = jnp.sum(slot * oh1, axis=1, keepdims=True).astype(jnp.int32)
    pos1_ref[...] = jnp.sum(slot * oh2, axis=1, keepdims=True).astype(jnp.int32)

    n_real = jnp.sum(padded_tiles)
    gi = jax.lax.broadcasted_iota(jnp.int32, (_CB, NUM_EXPERTS), 0)
    eid = jnp.minimum(
        jnp.sum(((gi - 1).astype(jnp.float32) >= bounds).astype(jnp.float32),
                axis=1, keepdims=True),
        NUM_EXPERTS - 1.0)
    l0 = jax.lax.broadcasted_iota(jnp.int32, (_CB, 1), 0)
    meta_ref[...] = jnp.where(l0 == 0, n_real, eid).astype(jnp.int32)


def _run_router(xt, router_w):
    T = xt.shape[0]
    shp = [jax.ShapeDtypeStruct((T, 1), jnp.int32),
           jax.ShapeDtypeStruct((T, 1), jnp.int32),
           jax.ShapeDtypeStruct((_CB, 1), jnp.int32)]
    return pl.pallas_call(_router_body, out_shape=shp)(xt, router_w)


@functools.cache
def _sc_mesh():
    return plsc.VectorSubcoreMesh(
        core_axis_name="c", subcore_axis_name="s",
        num_cores=NC, num_subcores=NS)


@functools.cache
def _make_sc_dispatch():
    @functools.partial(
        pl.kernel, mesh=_sc_mesh(),
        out_type=jax.ShapeDtypeStruct((NP, D_MODEL), jnp.float32),
        scratch_types=[
            pltpu.VMEM((TOK_W,), jnp.int32),
            pltpu.VMEM((TOK_W,), jnp.int32),
            pltpu.VMEM((TOK_W, D_MODEL), jnp.float32),
            pltpu.SemaphoreType.DMA,
            pltpu.SemaphoreType.DMA,
        ],
    )
    def _sc_dispatch(x_hbm, pos0_hbm, pos1_hbm, xs_hbm, i0, i1, xbuf, s0, s1):
        wid = lax.axis_index("s") * NC + lax.axis_index("c")
        t0 = wid * TOK_W
        pltpu.sync_copy(pos0_hbm.at[pl.ds(t0, TOK_W)], i0)
        pltpu.sync_copy(pos1_hbm.at[pl.ds(t0, TOK_W)], i1)
        pltpu.sync_copy(x_hbm.at[pl.ds(t0, TOK_W)], xbuf)
        c0 = pltpu.async_copy(xbuf, xs_hbm.at[i0], s0)
        c1 = pltpu.async_copy(xbuf, xs_hbm.at[i1], s1)
        c0.wait()
        c1.wait()

    return _sc_dispatch


def _dispatch_rows(xt, pos0, pos1):
    return _make_sc_dispatch()(xt, pos0, pos1)


def _ffn_body(m_ref, xs_ref, rw_ref, w1_ref, w2_ref, w3_ref, ys_ref):
    pid = pl.program_id(0)

    @pl.when(pid < m_ref[0])
    def _():
        e = m_ref[1 + pid]
        xt = xs_ref[...]
        i1, i2, p1, p2 = _top2(xt, rw_ref[...])
        w = p1 * (i1 == e) + p2 * (i2 == e)
        a = _dot_t(xt, w1_ref[0])
        b = _dot_t(xt, w3_ref[0])
        h = (a * jax.nn.sigmoid(a)) * b
        ys_ref[...] = _dot_t(h, w2_ref[0]) * w


def _run_ffn(meta, xs, router_w, W1, W2, W3):
    grid_spec = pltpu.PrefetchScalarGridSpec(
        num_scalar_prefetch=1,
        grid=(G_MAX,),
        in_specs=[
            pl.BlockSpec((TILE, D_MODEL), lambda g, m: (g, 0)),
            pl.BlockSpec((NUM_EXPERTS, D_MODEL), lambda g, m: (0, 0)),
            pl.BlockSpec((1, D_FF, D_MODEL), lambda g, m: (m[1 + g], 0, 0)),
            pl.BlockSpec((1, D_MODEL, D_FF), lambda g, m: (m[1 + g], 0, 0)),
            pl.BlockSpec((1, D_FF, D_MODEL), lambda g, m: (m[1 + g], 0, 0)),
        ],
        out_specs=pl.BlockSpec((TILE, D_MODEL), lambda g, m: (g, 0)),
    )
    return pl.pallas_call(
        _ffn_body,
        grid_spec=grid_spec,
        out_shape=jax.ShapeDtypeStruct((NP, D_MODEL), jnp.float32),
        compiler_params=pltpu.CompilerParams(vmem_limit_bytes=100 * 1024 * 1024),
    )(meta, xs, router_w, W1, W2, W3)


@functools.cache
def _make_sc_combine():
    @functools.partial(
        pl.kernel, mesh=_sc_mesh(),
        out_type=jax.ShapeDtypeStruct((T_TOKENS, D_MODEL), jnp.float32),
        scratch_types=[
            pltpu.VMEM((TOK_W,), jnp.int32),
            pltpu.VMEM((TOK_W,), jnp.int32),
            pltpu.VMEM((TOK_W, D_MODEL), jnp.float32),
            pltpu.VMEM((TOK_W, D_MODEL), jnp.float32),
            pltpu.SemaphoreType.DMA,
            pltpu.SemaphoreType.DMA,
            pltpu.SemaphoreType.DMA,
            pltpu.SemaphoreType.DMA,
        ],
    )
    def _sc_combine(ys_hbm, pos0_hbm, pos1_hbm, out_hbm, i0, i1, b0, b1,
                    s00, s01, s10, s11):
        wid = lax.axis_index("s") * NC + lax.axis_index("c")
        t0 = wid * TOK_W
        H = TOK_W // 2
        pltpu.sync_copy(pos0_hbm.at[pl.ds(t0, TOK_W)], i0)
        pltpu.sync_copy(pos1_hbm.at[pl.ds(t0, TOK_W)], i1)
        c00 = pltpu.async_copy(ys_hbm.at[i0.at[pl.ds(0, H)]], b0.at[pl.ds(0, H)], s00)
        c01 = pltpu.async_copy(ys_hbm.at[i1.at[pl.ds(0, H)]], b1.at[pl.ds(0, H)], s01)
        c10 = pltpu.async_copy(ys_hbm.at[i0.at[pl.ds(H, H)]], b0.at[pl.ds(H, H)], s10)
        c11 = pltpu.async_copy(ys_hbm.at[i1.at[pl.ds(H, H)]], b1.at[pl.ds(H, H)], s11)

        def _row(r, carry):
            for c in range(D_MODEL // 16):
                sl = pl.ds(c * 16, 16)
                b0[r, sl] = b0[r, sl] + b1[r, sl]
            return carry

        c00.wait()
        c01.wait()
        lax.fori_loop(0, H, _row, 0)
        pltpu.sync_copy(b0.at[pl.ds(0, H)], out_hbm.at[pl.ds(t0, H)])
        c10.wait()
        c11.wait()
        lax.fori_loop(H, TOK_W, _row, 0)
        pltpu.sync_copy(b0.at[pl.ds(H, H)], out_hbm.at[pl.ds(t0 + H, H)])

    return _sc_combine


def _combine_rows(ys, pos0, pos1):
    return _make_sc_combine()(ys, pos0, pos1)


def kernel(x, router_w, W1, W2, W3):
    B, S, D = x.shape
    T = B * S
    xt = x.reshape(T, D)

    pos0, pos1, meta_col = _run_router(xt, router_w)
    pos0 = pos0.reshape(T)
    pos1 = pos1.reshape(T)
    meta = meta_col[:G_MAX + 1, 0]

    xs = _dispatch_rows(xt, pos0, pos1)
    ys = _run_ffn(meta, xs, router_w, W1, W2, W3)
    out = _combine_rows(ys, pos0, pos1)
    return out.reshape(B, S, D)


# speedup vs baseline: 1.0454x; 1.0219x over previous
import functools

import jax
import jax.numpy as jnp
from jax import lax
from jax.experimental import pallas as pl
from jax.experimental.pallas import tpu as pltpu
from jax.experimental.pallas import tpu_sc as plsc

D_MODEL = 768
NUM_EXPERTS = 8
D_FF = 2048
TOP_K = 2
T_TOKENS = 2048

TILE = 512
N_PAIRS = T_TOKENS * TOP_K
G_MAX = N_PAIRS // TILE + NUM_EXPERTS
NP = G_MAX * TILE
NC, NS = 2, 16
NW = NC * NS
TOK_W = T_TOKENS // NW


def _dot_t(a, b):
    return jax.lax.dot_general(
        a, b, dimension_numbers=(((1,), (1,)), ((), ())),
        preferred_element_type=jnp.float32)


def _top2(xt, rw):
    logits = _dot_t(xt, rw)
    ii = jax.lax.broadcasted_iota(jnp.int32, logits.shape, 1)
    m1 = jnp.max(logits, axis=1, keepdims=True)
    i1 = jnp.min(jnp.where(logits == m1, ii, NUM_EXPERTS), axis=1, keepdims=True)
    l2 = jnp.where(ii == i1, -jnp.inf, logits)
    m2 = jnp.max(l2, axis=1, keepdims=True)
    i2 = jnp.min(jnp.where(l2 == m2, ii, NUM_EXPERTS), axis=1, keepdims=True)
    r = jnp.exp(m2 - m1)
    p1 = 1.0 / (1.0 + r)
    return i1, i2, p1, r * p1


_CB = 128
_NB = T_TOKENS // _CB


def _router_body(x_ref, rw_ref, pos0_ref, pos1_ref, meta_ref):
    i1, i2, _, _ = _top2(x_ref[...], rw_ref[...])
    er = jax.lax.broadcasted_iota(jnp.int32, (T_TOKENS, NUM_EXPERTS), 1)
    oh1 = (i1 == er).astype(jnp.float32)
    oh2 = (i2 == er).astype(jnp.float32)
    both = oh1 + oh2

    row = jax.lax.broadcasted_iota(jnp.int32, (_CB, _CB), 0)
    col = jax.lax.broadcasted_iota(jnp.int32, (_CB, _CB), 1)
    stril = (row > col).astype(jnp.float32)
    carry = jnp.zeros((1, NUM_EXPERTS), jnp.float32)
    parts = []
    for b in range(_NB):
        blk = both[b * _CB:(b + 1) * _CB, :]
        parts.append(jax.lax.dot(stril, blk,
                                 precision=jax.lax.Precision.DEFAULT) + carry)
        carry = carry + jnp.sum(blk, axis=0, keepdims=True)
    excl = jnp.concatenate(parts, axis=0)

    counts = carry
    padded_tiles = jnp.floor((counts + (TILE - 1)) * (1.0 / TILE))
    r8 = jax.lax.broadcasted_iota(jnp.int32, (NUM_EXPERTS, NUM_EXPERTS), 0)
    c8 = jax.lax.broadcasted_iota(jnp.int32, (NUM_EXPERTS, NUM_EXPERTS), 1)
    triu = (r8 <= c8).astype(jnp.float32)
    bounds = jax.lax.dot(padded_tiles, triu,
                         precision=jax.lax.Precision.DEFAULT)
    pad_off = (bounds - padded_tiles) * TILE
    slot = excl + pad_off
    pos0_ref[...] = jnp.sum(slot * oh1, axis=1).astype(jnp.int32)
    pos1_ref[...] = jnp.sum(slot * oh2, axis=1).astype(jnp.int32)

    n_real = jnp.sum(padded_tiles)
    gi = jax.lax.broadcasted_iota(jnp.int32, (_CB, NUM_EXPERTS), 0)
    eid = jnp.minimum(
        jnp.sum(((gi - 1).astype(jnp.float32) >= bounds).astype(jnp.float32),
                axis=1, keepdims=True),
        NUM_EXPERTS - 1.0)
    l0 = jax.lax.broadcasted_iota(jnp.int32, (_CB, 1), 0)
    meta_ref[...] = jnp.where(l0 == 0, n_real, eid).astype(jnp.int32)


def _run_router(xt, router_w):
    T = xt.shape[0]
    shp = [jax.ShapeDtypeStruct((T,), jnp.int32),
           jax.ShapeDtypeStruct((T,), jnp.int32),
           jax.ShapeDtypeStruct((_CB, 1), jnp.int32)]
    return pl.pallas_call(_router_body, out_shape=shp)(xt, router_w)


@functools.cache
def _sc_mesh():
    return plsc.VectorSubcoreMesh(
        core_axis_name="c", subcore_axis_name="s",
        num_cores=NC, num_subcores=NS)


@functools.cache
def _make_sc_dispatch():
    @functools.partial(
        pl.kernel, mesh=_sc_mesh(),
        out_type=jax.ShapeDtypeStruct((NP, D_MODEL), jnp.float32),
        scratch_types=[
            pltpu.VMEM((TOK_W,), jnp.int32),
            pltpu.VMEM((TOK_W,), jnp.int32),
            pltpu.VMEM((TOK_W, D_MODEL), jnp.float32),
            pltpu.SemaphoreType.DMA,
            pltpu.SemaphoreType.DMA,
        ],
    )
    def _sc_dispatch(x_hbm, pos0_hbm, pos1_hbm, xs_hbm, i0, i1, xbuf, s0, s1):
        wid = lax.axis_index("s") * NC + lax.axis_index("c")
        t0 = wid * TOK_W
        pltpu.sync_copy(pos0_hbm.at[pl.ds(t0, TOK_W)], i0)
        pltpu.sync_copy(pos1_hbm.at[pl.ds(t0, TOK_W)], i1)
        pltpu.sync_copy(x_hbm.at[pl.ds(t0, TOK_W)], xbuf)
        c0 = pltpu.async_copy(xbuf, xs_hbm.at[i0], s0)
        c1 = pltpu.async_copy(xbuf, xs_hbm.at[i1], s1)
        c0.wait()
        c1.wait()

    return _sc_dispatch


def _dispatch_rows(xt, pos0, pos1):
    return _make_sc_dispatch()(xt, pos0, pos1)


def _ffn_body(m_ref, xs_ref, rw_ref, w1_ref, w2_ref, w3_ref, ys_ref):
    pid = pl.program_id(0)

    @pl.when(pid < m_ref[0])
    def _():
        e = m_ref[1 + pid]
        xt = xs_ref[...]
        i1, i2, p1, p2 = _top2(xt, rw_ref[...])
        w = p1 * (i1 == e) + p2 * (i2 == e)
        a = _dot_t(xt, w1_ref[0])
        b = _dot_t(xt, w3_ref[0])
        h = (a * jax.nn.sigmoid(a)) * b
        ys_ref[...] = _dot_t(h, w2_ref[0]) * w


def _run_ffn(meta, xs, router_w, W1, W2, W3):
    grid_spec = pltpu.PrefetchScalarGridSpec(
        num_scalar_prefetch=1,
        grid=(G_MAX,),
        in_specs=[
            pl.BlockSpec((TILE, D_MODEL), lambda g, m: (g, 0)),
            pl.BlockSpec((NUM_EXPERTS, D_MODEL), lambda g, m: (0, 0)),
            pl.BlockSpec((1, D_FF, D_MODEL), lambda g, m: (m[1 + g], 0, 0)),
            pl.BlockSpec((1, D_MODEL, D_FF), lambda g, m: (m[1 + g], 0, 0)),
            pl.BlockSpec((1, D_FF, D_MODEL), lambda g, m: (m[1 + g], 0, 0)),
        ],
        out_specs=pl.BlockSpec((TILE, D_MODEL), lambda g, m: (g, 0)),
    )
    return pl.pallas_call(
        _ffn_body,
        grid_spec=grid_spec,
        out_shape=jax.ShapeDtypeStruct((NP, D_MODEL), jnp.float32),
        compiler_params=pltpu.CompilerParams(vmem_limit_bytes=100 * 1024 * 1024),
    )(meta, xs, router_w, W1, W2, W3)


@functools.cache
def _make_sc_combine():
    @functools.partial(
        pl.kernel, mesh=_sc_mesh(),
        out_type=jax.ShapeDtypeStruct((T_TOKENS, D_MODEL), jnp.float32),
        scratch_types=[
            pltpu.VMEM((TOK_W,), jnp.int32),
            pltpu.VMEM((TOK_W,), jnp.int32),
            pltpu.VMEM((TOK_W, D_MODEL), jnp.float32),
            pltpu.VMEM((TOK_W, D_MODEL), jnp.float32),
            pltpu.SemaphoreType.DMA,
            pltpu.SemaphoreType.DMA,
            pltpu.SemaphoreType.DMA,
            pltpu.SemaphoreType.DMA,
        ],
    )
    def _sc_combine(ys_hbm, pos0_hbm, pos1_hbm, out_hbm, i0, i1, b0, b1,
                    s00, s01, s10, s11):
        wid = lax.axis_index("s") * NC + lax.axis_index("c")
        t0 = wid * TOK_W
        H = TOK_W // 2
        pltpu.sync_copy(pos0_hbm.at[pl.ds(t0, TOK_W)], i0)
        pltpu.sync_copy(pos1_hbm.at[pl.ds(t0, TOK_W)], i1)
        c00 = pltpu.async_copy(ys_hbm.at[i0.at[pl.ds(0, H)]], b0.at[pl.ds(0, H)], s00)
        c01 = pltpu.async_copy(ys_hbm.at[i1.at[pl.ds(0, H)]], b1.at[pl.ds(0, H)], s01)
        c10 = pltpu.async_copy(ys_hbm.at[i0.at[pl.ds(H, H)]], b0.at[pl.ds(H, H)], s10)
        c11 = pltpu.async_copy(ys_hbm.at[i1.at[pl.ds(H, H)]], b1.at[pl.ds(H, H)], s11)

        def _row(r, carry):
            for c in range(D_MODEL // 16):
                sl = pl.ds(c * 16, 16)
                b0[r, sl] = b0[r, sl] + b1[r, sl]
            return carry

        c00.wait()
        c01.wait()
        lax.fori_loop(0, H, _row, 0)
        pltpu.sync_copy(b0.at[pl.ds(0, H)], out_hbm.at[pl.ds(t0, H)])
        c10.wait()
        c11.wait()
        lax.fori_loop(H, TOK_W, _row, 0)
        pltpu.sync_copy(b0.at[pl.ds(H, H)], out_hbm.at[pl.ds(t0 + H, H)])

    return _sc_combine


def _combine_rows(ys, pos0, pos1):
    return _make_sc_combine()(ys, pos0, pos1)


def kernel(x, router_w, W1, W2, W3):
    B, S, D = x.shape
    T = B * S
    xt = x.reshape(T, D)

    pos0, pos1, meta_col = _run_router(xt, router_w)
    meta = meta_col[:G_MAX + 1, 0]

    xs = _dispatch_rows(xt, pos0, pos1)
    ys = _run_ffn(meta, xs, router_w, W1, W2, W3)
    out = _combine_rows(ys, pos0, pos1)
    return out.reshape(B, S, D)
